# Initial kernel scaffold; baseline (speedup 1.0000x reference)
import jax
import jax.numpy as jnp
from jax.experimental import pallas as pl


def kernel(node_feats, edge_index, Wq, Wk, Wv):
    raise NotImplementedError("write your pallas kernel here")


# speedup vs baseline: 7.6099x; 7.6099x over previous
import functools

import jax
import jax.numpy as jnp
from jax import lax
from jax.experimental import pallas as pl
from jax.experimental.pallas import tpu as pltpu
from jax.experimental.pallas import tpu_sc as plsc

N = 10000
E = 320000
D = 128
NC, NS, L = 2, 16, 16
W = NC * NS
EPW = E // W
B = 80
CHUNKS = EPW // B
GPB = B // L
ACC_W = 144
RPS = N // NS
ZR = 125
NB_D = D // L


def _qkv_body(x_ref, wq_ref, wk_ref, wv_ref, q_ref, k_ref, v_ref):
    x = x_ref[...]
    inv_scale = 1.0 / (float(D) ** 0.5)
    q_ref[...] = jnp.dot(x, wq_ref[...], preferred_element_type=jnp.float32) * inv_scale
    k_ref[...] = jnp.dot(x, wk_ref[...], preferred_element_type=jnp.float32)
    v_ref[...] = jnp.dot(x, wv_ref[...], preferred_element_type=jnp.float32)


def _qkv(x, wq, wk, wv):
    blk = 1000
    grid = N // blk
    w_spec = pl.BlockSpec((D, D), lambda i: (0, 0))
    x_spec = pl.BlockSpec((blk, D), lambda i: (i, 0))
    out = jax.ShapeDtypeStruct((N, D), jnp.float32)
    return pl.pallas_call(
        _qkv_body,
        grid=(grid,),
        in_specs=[x_spec, w_spec, w_spec, w_spec],
        out_specs=[x_spec, x_spec, x_spec],
        out_shape=[out, out, out],
    )(x, wq, wk, wv)


def _sc_body(q_hbm, k_hbm, v_hbm, ei_hbm, hp_hbm,
             dst_v, src_v, qrows, kvrows, outb, accs, wsc, sem, h_sh):
    cid = lax.axis_index("c")
    sid = lax.axis_index("s")
    wid = sid * NC + cid
    lane = lax.iota(jnp.int32, L)
    zero16 = jnp.zeros((L,), jnp.float32)

    def zrow(r, _):
        for j in range(ACC_W // L):
            outb[r, pl.ds(j * L, L)] = zero16
        return 0
    lax.fori_loop(0, B, zrow, 0)
    for t in range(RPS // B):
        pltpu.sync_copy(outb, h_sh.at[pl.ds(sid * RPS + t * B, B)])
    rem = RPS - (RPS // B) * B
    if rem:
        pltpu.sync_copy(outb.at[pl.ds(0, rem)],
                        h_sh.at[pl.ds(sid * RPS + (RPS // B) * B, rem)])
    plsc.subcore_barrier()

    def chunk(ci, _):
        base = wid * EPW + ci * B
        pltpu.sync_copy(ei_hbm.at[1, pl.ds(base, B)], dst_v)
        pltpu.sync_copy(ei_hbm.at[0, pl.ds(base, B)], src_v)
        cq = pltpu.async_copy(q_hbm.at[dst_v], qrows, sem)
        ck = pltpu.async_copy(k_hbm.at[src_v], kvrows, sem)
        cq.wait()
        ck.wait()

        def group(g, _):
            def edot(e16, _):
                e = g * L + e16
                acc = qrows[e, pl.ds(0, L)] * kvrows[e, pl.ds(0, L)]
                for j in range(1, NB_D):
                    acc = acc + qrows[e, pl.ds(j * L, L)] * kvrows[e, pl.ds(j * L, L)]
                accs[pl.ds((g * L + e16) * L, L)] = acc
                return 0
            lax.fori_loop(0, L, edot, 0)
            return 0
        lax.fori_loop(0, GPB, group, 0)

        cv = pltpu.async_copy(v_hbm.at[src_v], kvrows, sem)

        for g in range(GPB):
            s = plsc.load_gather(accs, [g * L * L + lane * L])
            for j in range(1, L):
                s = s + plsc.load_gather(accs, [g * L * L + lane * L + j])
            w = jnp.exp(jnp.minimum(s, 70.0))
            wsc[pl.ds(g * L, L)] = w
        cv.wait()

        def vw(e, _):
            wv = plsc.load_gather(wsc, [jnp.full((L,), e, jnp.int32)])
            for j in range(NB_D):
                outb[e, pl.ds(j * L, L)] = kvrows[e, pl.ds(j * L, L)] * wv
            outb[e, pl.ds(D, L)] = jnp.where(lane == 0, wv, 0.0)
            return 0
        lax.fori_loop(0, B, vw, 0)

        pltpu.sync_copy(outb, h_sh.at[dst_v], add=True)
        return 0
    lax.fori_loop(0, CHUNKS, chunk, 0)

    plsc.subcore_barrier()
    pltpu.sync_copy(h_sh.at[pl.ds(sid * RPS, RPS)],
                    hp_hbm.at[cid, pl.ds(sid * RPS, RPS)])


_sc_attention = functools.partial(
    pl.kernel,
    out_type=jax.ShapeDtypeStruct((NC, N, ACC_W), jnp.float32),
    mesh=plsc.VectorSubcoreMesh(core_axis_name="c", subcore_axis_name="s"),
    compiler_params=pltpu.CompilerParams(
        use_tc_tiling_on_sc=False, needs_layout_passes=False),
    scratch_types=[
        pltpu.VMEM((B,), jnp.int32),
        pltpu.VMEM((B,), jnp.int32),
        pltpu.VMEM((B, D), jnp.float32),
        pltpu.VMEM((B, D), jnp.float32),
        pltpu.VMEM((B, ACC_W), jnp.float32),
        pltpu.VMEM((B * L,), jnp.float32),
        pltpu.VMEM((B,), jnp.float32),
        pltpu.SemaphoreType.DMA,
        pltpu.VMEM_SHARED((N, ACC_W), jnp.float32),
    ],
)(_sc_body)


def _combine_body(hp_ref, h_ref):
    s = hp_ref[0] + hp_ref[1]
    denom = jnp.maximum(s[:, D:D + 1], 1e-16)
    h_ref[...] = s[:, :D] / denom


def _combine(hp):
    blk = 1000
    return pl.pallas_call(
        _combine_body,
        grid=(N // blk,),
        in_specs=[pl.BlockSpec((NC, blk, ACC_W), lambda i: (0, i, 0))],
        out_specs=pl.BlockSpec((blk, D), lambda i: (i, 0)),
        out_shape=jax.ShapeDtypeStruct((N, D), jnp.float32),
    )(hp)


def kernel(node_feats, edge_index, Wq, Wk, Wv):
    q, k, v = _qkv(node_feats, Wq, Wk, Wv)
    hp = _sc_attention(q, k, v, edge_index)
    return _combine(hp)
